# baseline (speedup 1.0000x reference)
import jax
import jax.numpy as jnp
from jax.experimental import pallas as pl
from jax.experimental.pallas import tpu as pltpu

N = 1024
HID = 64
UCHUNK = 8
RHSROWS = UCHUNK + HID

_DN_T = (((1,), (1,)), ((), ()))
_DN = (((1,), (0,)), ((), ()))


def _mpnn_kernel(scal_ref, A_ref, E_ref, X_ref, encWT_ref, encb_ref,
                 Wh_ref, Lsel_ref, msgbcol_ref, WihT_ref, WhhT_ref,
                 bih_ref, bhh_ref, ro1WT_ref, ro1b_ref, ro2WT_ref, ro2b_ref,
                 out_ref, h_ref, rhs_ref, msgs_ref, invd_ref, mask_ref):
    h0 = jnp.tanh(
        jax.lax.dot(X_ref[...], encWT_ref[...],
                    preferred_element_type=jnp.float32) + encb_ref[...])
    h_ref[...] = h0

    maskf = (A_ref[...] > 0.0).astype(jnp.float32)
    mask_ref[...] = maskf
    deg = jnp.sum(maskf, axis=1, keepdims=True)
    invd_ref[...] = jnp.where(deg > 0.0, 1.0 / jnp.maximum(deg, 1.0), 0.0)

    def step_body(_, carry):
        h = h_ref[...]
        hmT = jax.lax.dot_general(Wh_ref[...], h, _DN_T,
                                  preferred_element_type=jnp.float32)
        rhs_ref[UCHUNK:, :] = hmT + msgbcol_ref[...]

        def u_body(j, c):
            u0 = j * UCHUNK
            rhs_ref[0:UCHUNK, :] = E_ref[pl.ds(u0, UCHUNK), :]
            rhs = rhs_ref[...]
            srows = []
            for i in range(UCHUNK):
                pre = jax.lax.dot_general(
                    Lsel_ref[i * HID:(i + 1) * HID, :], rhs, _DN,
                    preferred_element_type=jnp.float32)
                T = jnp.tanh(pre)
                mrow = mask_ref[pl.ds(u0 + i, 1), :]
                s = jax.lax.dot_general(mrow, T, _DN_T,
                                        preferred_element_type=jnp.float32)
                srows.append(s)
            blk = jnp.concatenate(srows, axis=0)
            msgs_ref[pl.ds(u0, UCHUNK), :] = blk * invd_ref[pl.ds(u0, UCHUNK), :]
            return c

        jax.lax.fori_loop(0, N // UCHUNK, u_body, 0, unroll=False)

        msgs = msgs_ref[...]
        gi = jax.lax.dot(msgs, WihT_ref[...],
                         preferred_element_type=jnp.float32) + bih_ref[...]
        gh = jax.lax.dot(h, WhhT_ref[...],
                         preferred_element_type=jnp.float32) + bhh_ref[...]
        r = jax.nn.sigmoid(gi[:, :HID] + gh[:, :HID])
        z = jax.nn.sigmoid(gi[:, HID:2 * HID] + gh[:, HID:2 * HID])
        ng = jnp.tanh(gi[:, 2 * HID:] + r * gh[:, 2 * HID:])
        h_ref[...] = (1.0 - z) * ng + z * h
        return carry

    jax.lax.fori_loop(0, scal_ref[0], step_body, 0)

    hs = h_ref[pl.ds(scal_ref[1], 1), :]
    ht = h_ref[pl.ds(scal_ref[2], 1), :]
    cat = jnp.concatenate([hs, ht], axis=1)
    mid = jax.nn.relu(
        jax.lax.dot(cat, ro1WT_ref[...],
                    preferred_element_type=jnp.float32) + ro1b_ref[...])
    out = jax.lax.dot(mid, ro2WT_ref[...],
                      preferred_element_type=jnp.float32) + ro2b_ref[...]
    out_ref[...] = jax.nn.sigmoid(out)


def kernel(A, E, X, enc_W, enc_b, msg_W, msg_b, W_ih, W_hh, b_ih, b_hh,
           ro1_W, ro1_b, ro2_W, ro2_b, source, target, steps):
    f32 = jnp.float32
    scal = jnp.stack([jnp.asarray(steps, jnp.int32),
                      jnp.asarray(source, jnp.int32),
                      jnp.asarray(target, jnp.int32)])
    hid = HID
    we = msg_W[:, hid].astype(f32)
    sel = jnp.eye(UCHUNK, dtype=f32)
    Lwe = we[None, :, None] * sel[:, None, :]
    Leye = jnp.broadcast_to(jnp.eye(hid, dtype=f32)[None], (UCHUNK, hid, hid))
    Lsel = jnp.concatenate([Lwe, Leye], axis=2).reshape(UCHUNK * hid, RHSROWS)
    args = (
        scal,
        A.astype(f32), E.astype(f32), X.astype(f32),
        enc_W.T.astype(f32), enc_b.reshape(1, hid).astype(f32),
        msg_W[:, :hid].astype(f32),
        Lsel,
        msg_b.reshape(hid, 1).astype(f32),
        W_ih.T.astype(f32), W_hh.T.astype(f32),
        b_ih.reshape(1, 3 * hid).astype(f32), b_hh.reshape(1, 3 * hid).astype(f32),
        ro1_W.T.astype(f32), ro1_b.reshape(1, hid).astype(f32),
        ro2_W.T.astype(f32), ro2_b.reshape(1, 1).astype(f32),
    )
    in_specs = [pl.BlockSpec(memory_space=pltpu.SMEM)] + \
               [pl.BlockSpec(memory_space=pltpu.VMEM)] * (len(args) - 1)
    out = pl.pallas_call(
        _mpnn_kernel,
        out_shape=jax.ShapeDtypeStruct((1, 1), f32),
        in_specs=in_specs,
        out_specs=pl.BlockSpec(memory_space=pltpu.VMEM),
        scratch_shapes=[
            pltpu.VMEM((N, HID), f32),
            pltpu.VMEM((RHSROWS, N), f32),
            pltpu.VMEM((N, HID), f32),
            pltpu.VMEM((N, 1), f32),
            pltpu.VMEM((N, N), f32),
        ],
    )(*args)
    return out.reshape(1)


# speedup vs baseline: 1.5683x; 1.5683x over previous
import jax
import jax.numpy as jnp
from jax.experimental import pallas as pl
from jax.experimental.pallas import tpu as pltpu

N = 1024
HID = 64
UCHUNK = 8

_DN_T = (((1,), (1,)), ((), ()))


def _mpnn_kernel(scal_ref, A_ref, E_ref, X_ref, encWT_ref, encb_ref,
                 Wh_ref, wecol_ref, msgbcol_ref, WihT_ref, WhhT_ref,
                 bih_ref, bhh_ref, ro1WT_ref, ro1b_ref, ro2WT_ref, ro2b_ref,
                 out_ref, h_ref, hmT_ref, msgs_ref, invd_ref, mask_ref):
    bf16 = jnp.bfloat16
    h0 = jnp.tanh(
        jax.lax.dot(X_ref[...], encWT_ref[...],
                    preferred_element_type=jnp.float32) + encb_ref[...])
    h_ref[...] = h0

    maskf = (A_ref[...] > 0.0).astype(jnp.float32)
    mask_ref[...] = maskf.astype(bf16)
    deg = jnp.sum(maskf, axis=1, keepdims=True)
    invd_ref[...] = jnp.where(deg > 0.0, 1.0 / jnp.maximum(deg, 1.0), 0.0)

    def step_body(_, carry):
        h = h_ref[...]
        hmT = jax.lax.dot_general(Wh_ref[...], h, _DN_T,
                                  preferred_element_type=jnp.float32)
        hmT_ref[...] = (hmT + msgbcol_ref[...]).astype(bf16)
        wcol = wecol_ref[...]

        def u_body(j, c):
            u0 = j * UCHUNK
            erows = E_ref[pl.ds(u0, UCHUNK), :]
            marows = mask_ref[pl.ds(u0, UCHUNK), :]
            hmTv = hmT_ref[...]
            srows = []
            for i in range(UCHUNK):
                T = jnp.tanh(hmTv + wcol * erows[i:i + 1, :])
                mrow = marows[i:i + 1, :]
                s = jax.lax.dot_general(mrow, T, _DN_T,
                                        preferred_element_type=jnp.float32)
                srows.append(s)
            blk = jnp.concatenate(srows, axis=0)
            msgs_ref[pl.ds(u0, UCHUNK), :] = blk * invd_ref[pl.ds(u0, UCHUNK), :]
            return c

        jax.lax.fori_loop(0, N // UCHUNK, u_body, 0, unroll=False)

        msgs = msgs_ref[...]
        gi = jax.lax.dot(msgs, WihT_ref[...],
                         preferred_element_type=jnp.float32) + bih_ref[...]
        gh = jax.lax.dot(h, WhhT_ref[...],
                         preferred_element_type=jnp.float32) + bhh_ref[...]
        r = jax.nn.sigmoid(gi[:, :HID] + gh[:, :HID])
        z = jax.nn.sigmoid(gi[:, HID:2 * HID] + gh[:, HID:2 * HID])
        ng = jnp.tanh(gi[:, 2 * HID:] + r * gh[:, 2 * HID:])
        h_ref[...] = (1.0 - z) * ng + z * h
        return carry

    jax.lax.fori_loop(0, scal_ref[0], step_body, 0)

    hs = h_ref[pl.ds(scal_ref[1], 1), :]
    ht = h_ref[pl.ds(scal_ref[2], 1), :]
    cat = jnp.concatenate([hs, ht], axis=1)
    mid = jax.nn.relu(
        jax.lax.dot(cat, ro1WT_ref[...],
                    preferred_element_type=jnp.float32) + ro1b_ref[...])
    out = jax.lax.dot(mid, ro2WT_ref[...],
                      preferred_element_type=jnp.float32) + ro2b_ref[...]
    out_ref[...] = jax.nn.sigmoid(out)


def kernel(A, E, X, enc_W, enc_b, msg_W, msg_b, W_ih, W_hh, b_ih, b_hh,
           ro1_W, ro1_b, ro2_W, ro2_b, source, target, steps):
    f32 = jnp.float32
    bf16 = jnp.bfloat16
    scal = jnp.stack([jnp.asarray(steps, jnp.int32),
                      jnp.asarray(source, jnp.int32),
                      jnp.asarray(target, jnp.int32)])
    hid = HID
    args = (
        scal,
        A.astype(f32), E.astype(bf16), X.astype(f32),
        enc_W.T.astype(f32), enc_b.reshape(1, hid).astype(f32),
        msg_W[:, :hid].astype(f32),
        msg_W[:, hid:hid + 1].astype(bf16),
        msg_b.reshape(hid, 1).astype(f32),
        W_ih.T.astype(f32), W_hh.T.astype(f32),
        b_ih.reshape(1, 3 * hid).astype(f32), b_hh.reshape(1, 3 * hid).astype(f32),
        ro1_W.T.astype(f32), ro1_b.reshape(1, hid).astype(f32),
        ro2_W.T.astype(f32), ro2_b.reshape(1, 1).astype(f32),
    )
    in_specs = [pl.BlockSpec(memory_space=pltpu.SMEM)] + \
               [pl.BlockSpec(memory_space=pltpu.VMEM)] * (len(args) - 1)
    out = pl.pallas_call(
        _mpnn_kernel,
        out_shape=jax.ShapeDtypeStruct((1, 1), f32),
        in_specs=in_specs,
        out_specs=pl.BlockSpec(memory_space=pltpu.VMEM),
        scratch_shapes=[
            pltpu.VMEM((N, HID), f32),
            pltpu.VMEM((HID, N), bf16),
            pltpu.VMEM((N, HID), f32),
            pltpu.VMEM((N, 1), f32),
            pltpu.VMEM((N, N), bf16),
        ],
    )(*args)
    return out.reshape(1)


# speedup vs baseline: 1.9550x; 1.2466x over previous
import jax
import jax.numpy as jnp
from jax.experimental import pallas as pl
from jax.experimental.pallas import tpu as pltpu

N = 1024
HID = 64
UCHUNK = 16

_DN_T = (((1,), (1,)), ((), ()))


def _mpnn_kernel(scal_ref, A_ref, E_ref, X_ref, encWT_ref, encb_ref,
                 Wh_ref, wecol_ref, msgbcol_ref, WihT_ref, WhhT_ref,
                 bih_ref, bhh_ref, ro1WT_ref, ro1b_ref, ro2WT_ref, ro2b_ref,
                 out_ref, h_ref, hmT_ref, msgs_ref, invd_ref, mask_ref):
    bf16 = jnp.bfloat16
    h0 = jnp.tanh(
        jax.lax.dot(X_ref[...], encWT_ref[...],
                    preferred_element_type=jnp.float32) + encb_ref[...])
    h_ref[...] = h0

    maskf = (A_ref[...] > 0.0).astype(jnp.float32)
    mask_ref[...] = maskf.astype(bf16)
    deg = jnp.sum(maskf, axis=1, keepdims=True)
    invd_ref[...] = jnp.where(deg > 0.0, 1.0 / jnp.maximum(deg, 1.0), 0.0)

    def step_body(_, carry):
        h = h_ref[...]
        hmT = jax.lax.dot_general(Wh_ref[...], h, _DN_T,
                                  preferred_element_type=jnp.float32)
        hmT_ref[...] = (hmT + msgbcol_ref[...]).astype(bf16)
        wcol = wecol_ref[...]

        def u_body(j, c):
            u0 = j * UCHUNK
            erows = E_ref[pl.ds(u0, UCHUNK), :]
            marows = mask_ref[pl.ds(u0, UCHUNK), :]
            hmTv = hmT_ref[...]
            srows = []
            for i in range(UCHUNK):
                T = jnp.tanh(hmTv + wcol * erows[i:i + 1, :])
                mrow = marows[i:i + 1, :]
                s = jax.lax.dot_general(mrow, T, _DN_T,
                                        preferred_element_type=jnp.float32)
                srows.append(s)
            blk = jnp.concatenate(srows, axis=0)
            msgs_ref[pl.ds(u0, UCHUNK), :] = blk * invd_ref[pl.ds(u0, UCHUNK), :]
            return c

        jax.lax.fori_loop(0, N // UCHUNK, u_body, 0, unroll=False)

        msgs = msgs_ref[...]
        gi = jax.lax.dot(msgs, WihT_ref[...],
                         preferred_element_type=jnp.float32) + bih_ref[...]
        gh = jax.lax.dot(h, WhhT_ref[...],
                         preferred_element_type=jnp.float32) + bhh_ref[...]
        r = jax.nn.sigmoid(gi[:, :HID] + gh[:, :HID])
        z = jax.nn.sigmoid(gi[:, HID:2 * HID] + gh[:, HID:2 * HID])
        ng = jnp.tanh(gi[:, 2 * HID:] + r * gh[:, 2 * HID:])
        h_ref[...] = (1.0 - z) * ng + z * h
        return carry

    jax.lax.fori_loop(0, scal_ref[0], step_body, 0)

    hs = h_ref[pl.ds(scal_ref[1], 1), :]
    ht = h_ref[pl.ds(scal_ref[2], 1), :]
    cat = jnp.concatenate([hs, ht], axis=1)
    mid = jax.nn.relu(
        jax.lax.dot(cat, ro1WT_ref[...],
                    preferred_element_type=jnp.float32) + ro1b_ref[...])
    out = jax.lax.dot(mid, ro2WT_ref[...],
                      preferred_element_type=jnp.float32) + ro2b_ref[...]
    out_ref[...] = jax.nn.sigmoid(out)


def kernel(A, E, X, enc_W, enc_b, msg_W, msg_b, W_ih, W_hh, b_ih, b_hh,
           ro1_W, ro1_b, ro2_W, ro2_b, source, target, steps):
    f32 = jnp.float32
    bf16 = jnp.bfloat16
    scal = jnp.stack([jnp.asarray(steps, jnp.int32),
                      jnp.asarray(source, jnp.int32),
                      jnp.asarray(target, jnp.int32)])
    hid = HID
    args = (
        scal,
        A.astype(f32), E.astype(bf16), X.astype(f32),
        enc_W.T.astype(f32), enc_b.reshape(1, hid).astype(f32),
        msg_W[:, :hid].astype(f32),
        msg_W[:, hid:hid + 1].astype(bf16),
        msg_b.reshape(hid, 1).astype(f32),
        W_ih.T.astype(f32), W_hh.T.astype(f32),
        b_ih.reshape(1, 3 * hid).astype(f32), b_hh.reshape(1, 3 * hid).astype(f32),
        ro1_W.T.astype(f32), ro1_b.reshape(1, hid).astype(f32),
        ro2_W.T.astype(f32), ro2_b.reshape(1, 1).astype(f32),
    )
    in_specs = [pl.BlockSpec(memory_space=pltpu.SMEM)] + \
               [pl.BlockSpec(memory_space=pltpu.VMEM)] * (len(args) - 1)
    out = pl.pallas_call(
        _mpnn_kernel,
        out_shape=jax.ShapeDtypeStruct((1, 1), f32),
        in_specs=in_specs,
        out_specs=pl.BlockSpec(memory_space=pltpu.VMEM),
        scratch_shapes=[
            pltpu.VMEM((N, HID), f32),
            pltpu.VMEM((HID, N), bf16),
            pltpu.VMEM((N, HID), f32),
            pltpu.VMEM((N, 1), f32),
            pltpu.VMEM((N, N), bf16),
        ],
    )(*args)
    return out.reshape(1)
